# baseline (speedup 1.0000x reference)
import functools

import jax
import jax.numpy as jnp
from jax import lax
from jax.experimental import pallas as pl
from jax.experimental.pallas import tpu as pltpu
from jax.experimental.pallas import tpu_sc as plsc

N_EMBED = 768
CLF_TOKEN = 40480
SEQ = 2048
NSEQ = 8
LANES = 16
FLAT = 2 * SEQ
SCAN_UNROLL = 4
DOT_UNROLL = 4


def _mc_head_body(flat_hbm, hid_hbm, w_hbm, b_hbm, out_hbm, stage_hbm,
                  ids_v, row_v, w_v, tmp_v, res_v, big_v, b_v):
    w = lax.axis_index("s")
    iota = lax.iota(jnp.int32, LANES)

    @pl.when(w < NSEQ)
    def _():
        pltpu.sync_copy(flat_hbm.at[w], ids_v)

        iota_f = iota.astype(jnp.float32)
        neg1 = jnp.full((LANES,), -1.0, jnp.float32)
        accs = [neg1] * SCAN_UNROLL

        def scan_body(i, accs):
            accs = list(accs)
            for u in range(SCAN_UNROLL):
                off = i * (SCAN_UNROLL * LANES) + u * LANES
                chunk = ids_v[pl.ds(off, LANES)]
                idxs = iota_f + jnp.float32(off)
                accs[u] = jnp.maximum(accs[u],
                                      jnp.where(chunk == CLF_TOKEN, idxs, -1.0))
            return tuple(accs)

        accs = lax.fori_loop(0, FLAT // (SCAN_UNROLL * LANES), scan_body,
                             tuple(accs))
        acc = jnp.maximum(jnp.maximum(accs[0], accs[1]),
                          jnp.maximum(accs[2], accs[3]))
        flat_pos = acc[0]
        for i in range(1, LANES):
            flat_pos = jnp.maximum(flat_pos, acc[i])
        pos = flat_pos.astype(jnp.int32) // 2

        pltpu.sync_copy(hid_hbm.at[w * SEQ + pos], row_v)
        pltpu.sync_copy(w_hbm, w_v)
        pltpu.sync_copy(b_hbm, b_v)

        zero = jnp.zeros((LANES,), jnp.float32)
        daccs = [zero] * DOT_UNROLL

        def dot_body(j, daccs):
            daccs = list(daccs)
            for u in range(DOT_UNROLL):
                off = j * (DOT_UNROLL * LANES) + u * LANES
                daccs[u] = daccs[u] + row_v[pl.ds(off, LANES)] * w_v[pl.ds(off, LANES)]
            return tuple(daccs)

        daccs = lax.fori_loop(0, N_EMBED // (DOT_UNROLL * LANES), dot_body,
                              tuple(daccs))
        dacc = (daccs[0] + daccs[1]) + (daccs[2] + daccs[3])
        dacc = dacc + b_v[...]
        logit = dacc[0]
        for i in range(1, LANES):
            logit = logit + dacc[i]

        tmp_v[...] = jnp.where(iota == w, logit, 0.0)
        pltpu.sync_copy(tmp_v, stage_hbm.at[w])

    plsc.subcore_barrier()

    @pl.when(w == 0)
    def _():
        pltpu.sync_copy(stage_hbm, big_v)
        g = big_v[0, :]
        for i in range(1, NSEQ):
            g = g + big_v[i, :]
        res_v[...] = g
        pltpu.sync_copy(res_v.at[pl.ds(0, NSEQ)], out_hbm)


@jax.jit
def _mc_head(flat, hid, w, b16):
    mesh = plsc.VectorSubcoreMesh(core_axis_name="c", subcore_axis_name="s",
                                  num_cores=1)
    f = functools.partial(
        pl.kernel,
        mesh=mesh,
        out_type=(jax.ShapeDtypeStruct((NSEQ,), jnp.float32),
                  jax.ShapeDtypeStruct((NSEQ, LANES), jnp.float32)),
        scratch_types=[
            pltpu.VMEM((FLAT,), jnp.int32),
            pltpu.VMEM((N_EMBED,), jnp.float32),
            pltpu.VMEM((N_EMBED,), jnp.float32),
            pltpu.VMEM((LANES,), jnp.float32),
            pltpu.VMEM((LANES,), jnp.float32),
            pltpu.VMEM((NSEQ, LANES), jnp.float32),
            pltpu.VMEM((LANES,), jnp.float32),
        ],
    )(_mc_head_body)
    return f(flat, hid, w, b16)


def kernel(hidden, inputs, W, b):
    n_batch, n_choices = inputs.shape[0], inputs.shape[1]
    flat = inputs.reshape(NSEQ, FLAT)
    hid = hidden.reshape(NSEQ * SEQ, N_EMBED)
    b16 = jnp.pad(b, (0, LANES - 1))
    out, _stage = _mc_head(flat, hid, W.reshape(N_EMBED), b16)
    return out.reshape(n_batch, n_choices)


# speedup vs baseline: 1.6570x; 1.6570x over previous
import functools

import jax
import jax.numpy as jnp
from jax import lax
from jax.experimental import pallas as pl
from jax.experimental.pallas import tpu as pltpu
from jax.experimental.pallas import tpu_sc as plsc

N_EMBED = 768
CLF_TOKEN = 40480
SEQ = 2048
NSEQ = 8
LANES = 16
SCAN_UNROLL = 4
DOT_UNROLL = 4


def _mc_head_body(ids_hbm, hid_hbm, w_hbm, b_hbm, out_hbm, stage_hbm,
                  ids_v, row_v, w_v, tmp_v, res_v, big_v, b_v):
    w = lax.axis_index("s")
    iota = lax.iota(jnp.int32, LANES)

    @pl.when(w < NSEQ)
    def _():
        pltpu.sync_copy(ids_hbm.at[w], ids_v)

        iota_f = iota.astype(jnp.float32)
        neg1 = jnp.full((LANES,), -1.0, jnp.float32)
        accs = [neg1] * SCAN_UNROLL

        def scan_body(i, accs):
            accs = list(accs)
            for u in range(SCAN_UNROLL):
                off = i * (SCAN_UNROLL * LANES) + u * LANES
                chunk = ids_v[pl.ds(off, LANES)]
                idxs = iota_f + jnp.float32(off)
                accs[u] = jnp.maximum(accs[u],
                                      jnp.where(chunk == CLF_TOKEN, idxs, -1.0))
            return tuple(accs)

        accs = lax.fori_loop(0, SEQ // (SCAN_UNROLL * LANES), scan_body,
                             tuple(accs))
        acc = jnp.maximum(jnp.maximum(accs[0], accs[1]),
                          jnp.maximum(accs[2], accs[3]))
        pos_f = acc[0]
        for i in range(1, LANES):
            pos_f = jnp.maximum(pos_f, acc[i])
        pos = pos_f.astype(jnp.int32)

        pltpu.sync_copy(hid_hbm.at[w * SEQ + pos], row_v)
        pltpu.sync_copy(w_hbm, w_v)
        pltpu.sync_copy(b_hbm, b_v)

        zero = jnp.zeros((LANES,), jnp.float32)
        daccs = [zero] * DOT_UNROLL

        def dot_body(j, daccs):
            daccs = list(daccs)
            for u in range(DOT_UNROLL):
                off = j * (DOT_UNROLL * LANES) + u * LANES
                daccs[u] = daccs[u] + row_v[pl.ds(off, LANES)] * w_v[pl.ds(off, LANES)]
            return tuple(daccs)

        daccs = lax.fori_loop(0, N_EMBED // (DOT_UNROLL * LANES), dot_body,
                              tuple(daccs))
        dacc = (daccs[0] + daccs[1]) + (daccs[2] + daccs[3])
        dacc = dacc + b_v[...]
        logit = dacc[0]
        for i in range(1, LANES):
            logit = logit + dacc[i]

        tmp_v[...] = jnp.where(iota == w, logit, 0.0)
        pltpu.sync_copy(tmp_v, stage_hbm.at[w])

    plsc.subcore_barrier()

    @pl.when(w == 0)
    def _():
        pltpu.sync_copy(stage_hbm, big_v)
        g = big_v[0, :]
        for i in range(1, NSEQ):
            g = g + big_v[i, :]
        res_v[...] = g
        pltpu.sync_copy(res_v.at[pl.ds(0, NSEQ)], out_hbm)


@jax.jit
def _mc_head(ids, hid, w, b16):
    mesh = plsc.VectorSubcoreMesh(core_axis_name="c", subcore_axis_name="s",
                                  num_cores=1)
    f = functools.partial(
        pl.kernel,
        mesh=mesh,
        out_type=(jax.ShapeDtypeStruct((NSEQ,), jnp.float32),
                  jax.ShapeDtypeStruct((NSEQ, LANES), jnp.float32)),
        scratch_types=[
            pltpu.VMEM((SEQ,), jnp.int32),
            pltpu.VMEM((N_EMBED,), jnp.float32),
            pltpu.VMEM((N_EMBED,), jnp.float32),
            pltpu.VMEM((LANES,), jnp.float32),
            pltpu.VMEM((LANES,), jnp.float32),
            pltpu.VMEM((NSEQ, LANES), jnp.float32),
            pltpu.VMEM((LANES,), jnp.float32),
        ],
    )(_mc_head_body)
    return f(ids, hid, w, b16)


def kernel(hidden, inputs, W, b):
    n_batch, n_choices = inputs.shape[0], inputs.shape[1]
    ids = inputs[..., 0].reshape(NSEQ, SEQ)
    hid = hidden.reshape(NSEQ * SEQ, N_EMBED)
    b16 = jnp.pad(b, (0, LANES - 1))
    out, _stage = _mc_head(ids, hid, W.reshape(N_EMBED), b16)
    return out.reshape(n_batch, n_choices)


# speedup vs baseline: 1.7350x; 1.0470x over previous
import functools

import jax
import jax.numpy as jnp
from jax import lax
from jax.experimental import pallas as pl
from jax.experimental.pallas import tpu as pltpu
from jax.experimental.pallas import tpu_sc as plsc

N_EMBED = 768
CLF_TOKEN = 40480
SEQ = 2048
NSEQ = 8
LANES = 16
SCAN_UNROLL = 4
DOT_UNROLL = 4


def _mc_head_body(ids_hbm, hid_hbm, w_hbm, b_hbm, out_hbm, stage_hbm,
                  ids_v, row_v, w_v, tmp_v, res_v, big_v, b_v,
                  sem0, sem1, sem2):
    w = lax.axis_index("s")
    iota = lax.iota(jnp.int32, LANES)

    HALF = SEQ // 2

    @pl.when(w < NSEQ)
    def _():
        c0 = pltpu.async_copy(ids_hbm.at[w, pl.ds(0, HALF)],
                              ids_v.at[pl.ds(0, HALF)], sem0)
        c1 = pltpu.async_copy(ids_hbm.at[w, pl.ds(HALF, HALF)],
                              ids_v.at[pl.ds(HALF, HALF)], sem1)
        cw = pltpu.async_copy(w_hbm, w_v, sem2)

        iota_f = iota.astype(jnp.float32)
        neg1 = jnp.full((LANES,), -1.0, jnp.float32)

        def make_scan(base):
            def scan_body(i, accs):
                accs = list(accs)
                for u in range(SCAN_UNROLL):
                    off = base + i * (SCAN_UNROLL * LANES) + u * LANES
                    chunk = ids_v[pl.ds(off, LANES)]
                    idxs = iota_f + jnp.float32(off)
                    accs[u] = jnp.maximum(
                        accs[u], jnp.where(chunk == CLF_TOKEN, idxs, -1.0))
                return tuple(accs)
            return scan_body

        n_it = HALF // (SCAN_UNROLL * LANES)
        c0.wait()
        accs = lax.fori_loop(0, n_it, make_scan(0), (neg1,) * SCAN_UNROLL)
        c1.wait()
        accs = lax.fori_loop(0, n_it, make_scan(HALF), accs)
        acc = jnp.maximum(jnp.maximum(accs[0], accs[1]),
                          jnp.maximum(accs[2], accs[3]))
        pos_f = acc[0]
        for i in range(1, LANES):
            pos_f = jnp.maximum(pos_f, acc[i])
        pos = pos_f.astype(jnp.int32)

        pltpu.sync_copy(hid_hbm.at[w * SEQ + pos], row_v)
        cw.wait()

        zero = jnp.zeros((LANES,), jnp.float32)
        daccs = [zero] * DOT_UNROLL

        def dot_body(j, daccs):
            daccs = list(daccs)
            for u in range(DOT_UNROLL):
                off = j * (DOT_UNROLL * LANES) + u * LANES
                daccs[u] = daccs[u] + row_v[pl.ds(off, LANES)] * w_v[pl.ds(off, LANES)]
            return tuple(daccs)

        daccs = lax.fori_loop(0, N_EMBED // (DOT_UNROLL * LANES), dot_body,
                              tuple(daccs))
        dacc = (daccs[0] + daccs[1]) + (daccs[2] + daccs[3])
        logit = dacc[0]
        for i in range(1, LANES):
            logit = logit + dacc[i]

        tmp_v[...] = jnp.where(iota == w, logit, 0.0)
        pltpu.sync_copy(tmp_v, stage_hbm.at[w])

    plsc.subcore_barrier()

    @pl.when(w == 0)
    def _():
        cs = pltpu.async_copy(stage_hbm, big_v, sem0)
        cb = pltpu.async_copy(b_hbm, b_v, sem2)
        cs.wait()
        cb.wait()
        g = big_v[0, :] + b_v[...]
        for i in range(1, NSEQ):
            g = g + big_v[i, :]
        res_v[...] = g
        pltpu.sync_copy(res_v.at[pl.ds(0, NSEQ)], out_hbm)


@jax.jit
def _mc_head(ids, hid, w, b16):
    mesh = plsc.VectorSubcoreMesh(core_axis_name="c", subcore_axis_name="s",
                                  num_cores=1)
    f = functools.partial(
        pl.kernel,
        mesh=mesh,
        out_type=(jax.ShapeDtypeStruct((NSEQ,), jnp.float32),
                  jax.ShapeDtypeStruct((NSEQ, LANES), jnp.float32)),
        scratch_types=[
            pltpu.VMEM((SEQ,), jnp.int32),
            pltpu.VMEM((N_EMBED,), jnp.float32),
            pltpu.VMEM((N_EMBED,), jnp.float32),
            pltpu.VMEM((LANES,), jnp.float32),
            pltpu.VMEM((LANES,), jnp.float32),
            pltpu.VMEM((NSEQ, LANES), jnp.float32),
            pltpu.VMEM((LANES,), jnp.float32),
            pltpu.SemaphoreType.DMA,
            pltpu.SemaphoreType.DMA,
            pltpu.SemaphoreType.DMA,
        ],
    )(_mc_head_body)
    return f(ids, hid, w, b16)


def kernel(hidden, inputs, W, b):
    n_batch, n_choices = inputs.shape[0], inputs.shape[1]
    ids = inputs[..., 0].reshape(NSEQ, SEQ)
    hid = hidden.reshape(NSEQ * SEQ, N_EMBED)
    b16 = jnp.broadcast_to(b, (LANES,))
    out, _stage = _mc_head(ids, hid, W.reshape(N_EMBED), b16)
    return out.reshape(n_batch, n_choices)


# speedup vs baseline: 1.7740x; 1.0225x over previous
import functools

import jax
import jax.numpy as jnp
from jax import lax
from jax.experimental import pallas as pl
from jax.experimental.pallas import tpu as pltpu
from jax.experimental.pallas import tpu_sc as plsc

N_EMBED = 768
CLF_TOKEN = 40480
SEQ = 2048
NSEQ = 8
LANES = 16
SCAN_UNROLL = 4
DOT_UNROLL = 4


def _mc_head_body(ids_hbm, hid_hbm, wb_hbm, out_hbm, stage_hbm,
                  ids_v, row_v, w_v, tmp_v, res_v, big_v,
                  sem0, sem1, sem2):
    w = lax.axis_index("s")
    iota = lax.iota(jnp.int32, LANES)

    HALF = SEQ // 2

    @pl.when(w < NSEQ)
    def _():
        c0 = pltpu.async_copy(ids_hbm.at[w, pl.ds(0, HALF)],
                              ids_v.at[pl.ds(0, HALF)], sem0)
        c1 = pltpu.async_copy(ids_hbm.at[w, pl.ds(HALF, HALF)],
                              ids_v.at[pl.ds(HALF, HALF)], sem1)
        cw = pltpu.async_copy(wb_hbm, w_v, sem2)

        iota_f = iota.astype(jnp.float32)
        neg1 = jnp.full((LANES,), -1.0, jnp.float32)

        def make_scan(base):
            def scan_body(i, accs):
                accs = list(accs)
                for u in range(SCAN_UNROLL):
                    off = base + i * (SCAN_UNROLL * LANES) + u * LANES
                    chunk = ids_v[pl.ds(off, LANES)]
                    idxs = iota_f + jnp.float32(off)
                    accs[u] = jnp.maximum(
                        accs[u], jnp.where(chunk == CLF_TOKEN, idxs, -1.0))
                return tuple(accs)
            return scan_body

        n_it = HALF // (SCAN_UNROLL * LANES)
        c0.wait()
        accs = lax.fori_loop(0, n_it, make_scan(0), (neg1,) * SCAN_UNROLL)
        c1.wait()
        accs = lax.fori_loop(0, n_it, make_scan(HALF), accs)
        acc = jnp.maximum(jnp.maximum(accs[0], accs[1]),
                          jnp.maximum(accs[2], accs[3]))
        pos_f = acc[0]
        for i in range(1, LANES):
            pos_f = jnp.maximum(pos_f, acc[i])
        pos = pos_f.astype(jnp.int32)

        pltpu.sync_copy(hid_hbm.at[w * SEQ + pos], row_v)
        cw.wait()

        zero = jnp.zeros((LANES,), jnp.float32)
        daccs = [zero] * DOT_UNROLL

        def dot_body(j, daccs):
            daccs = list(daccs)
            for u in range(DOT_UNROLL):
                off = j * (DOT_UNROLL * LANES) + u * LANES
                daccs[u] = daccs[u] + row_v[pl.ds(off, LANES)] * w_v[pl.ds(off, LANES)]
            return tuple(daccs)

        daccs = lax.fori_loop(0, N_EMBED // (DOT_UNROLL * LANES), dot_body,
                              tuple(daccs))
        dacc = (daccs[0] + daccs[1]) + (daccs[2] + daccs[3])
        logit = dacc[0]
        for i in range(1, LANES):
            logit = logit + dacc[i]

        tmp_v[...] = jnp.where(iota == w, logit, 0.0)
        pltpu.sync_copy(tmp_v, stage_hbm.at[w])

    plsc.subcore_barrier()

    @pl.when(w == 0)
    def _():
        pltpu.sync_copy(stage_hbm, big_v)
        g = big_v[0, :] + w_v[pl.ds(N_EMBED, LANES)]
        for i in range(1, NSEQ):
            g = g + big_v[i, :]
        res_v[...] = g
        pltpu.sync_copy(res_v.at[pl.ds(0, NSEQ)], out_hbm)


@jax.jit
def _mc_head(ids, hid, wb):
    mesh = plsc.VectorSubcoreMesh(core_axis_name="c", subcore_axis_name="s",
                                  num_cores=1)
    f = functools.partial(
        pl.kernel,
        mesh=mesh,
        out_type=(jax.ShapeDtypeStruct((NSEQ,), jnp.float32),
                  jax.ShapeDtypeStruct((NSEQ, LANES), jnp.float32)),
        scratch_types=[
            pltpu.VMEM((SEQ,), jnp.int32),
            pltpu.VMEM((N_EMBED,), jnp.float32),
            pltpu.VMEM((N_EMBED + LANES,), jnp.float32),
            pltpu.VMEM((LANES,), jnp.float32),
            pltpu.VMEM((LANES,), jnp.float32),
            pltpu.VMEM((NSEQ, LANES), jnp.float32),
            pltpu.SemaphoreType.DMA,
            pltpu.SemaphoreType.DMA,
            pltpu.SemaphoreType.DMA,
        ],
    )(_mc_head_body)
    return f(ids, hid, wb)


def kernel(hidden, inputs, W, b):
    n_batch, n_choices = inputs.shape[0], inputs.shape[1]
    ids = inputs.reshape(NSEQ, SEQ, 2)[:, :, 0]
    hid = hidden.reshape(NSEQ * SEQ, N_EMBED)
    wb = jnp.concatenate([W[:, 0], jnp.broadcast_to(b, (LANES,))])
    out, _stage = _mc_head(ids, hid, wb)
    return out.reshape(n_batch, n_choices)


# speedup vs baseline: 1.7964x; 1.0126x over previous
import functools

import jax
import jax.numpy as jnp
from jax import lax
from jax.experimental import pallas as pl
from jax.experimental.pallas import tpu as pltpu
from jax.experimental.pallas import tpu_sc as plsc

N_EMBED = 768
CLF_TOKEN = 40480
SEQ = 2048
NSEQ = 8
LANES = 16
SCAN_UNROLL = 4
DOT_UNROLL = 4


def _mc_head_body(ids_hbm, hid_hbm, wb_hbm, out_hbm, stage_hbm,
                  ids_v, row_v, w_v, tmp_v, res_v, big_v,
                  sem0, sem1, sem2):
    w = lax.axis_index("s")
    iota = lax.iota(jnp.int32, LANES)

    HALF = SEQ // 2

    @pl.when(w < NSEQ)
    def _():
        c0 = pltpu.async_copy(ids_hbm.at[w, pl.ds(0, HALF)],
                              ids_v.at[pl.ds(0, HALF)], sem0)
        c1 = pltpu.async_copy(ids_hbm.at[w, pl.ds(HALF, HALF)],
                              ids_v.at[pl.ds(HALF, HALF)], sem1)
        cw = pltpu.async_copy(wb_hbm, w_v, sem2)

        iota_f = iota.astype(jnp.float32)
        neg1 = jnp.full((LANES,), -1.0, jnp.float32)

        def make_scan(base):
            def scan_body(i, accs):
                accs = list(accs)
                for u in range(SCAN_UNROLL):
                    off = base + i * (SCAN_UNROLL * LANES) + u * LANES
                    chunk = ids_v[pl.ds(off, LANES)]
                    idxs = iota_f + jnp.float32(off)
                    accs[u] = jnp.maximum(
                        accs[u], jnp.where(chunk == CLF_TOKEN, idxs, -1.0))
                return tuple(accs)
            return scan_body

        n_it = HALF // (SCAN_UNROLL * LANES)
        c0.wait()
        accs = lax.fori_loop(0, n_it, make_scan(0), (neg1,) * SCAN_UNROLL)
        c1.wait()
        accs = lax.fori_loop(0, n_it, make_scan(HALF), accs)
        acc = jnp.maximum(jnp.maximum(accs[0], accs[1]),
                          jnp.maximum(accs[2], accs[3]))
        pos_f = acc[0]
        for i in range(1, LANES):
            pos_f = jnp.maximum(pos_f, acc[i])
        pos = pos_f.astype(jnp.int32)

        pltpu.sync_copy(hid_hbm.at[w * SEQ + pos], row_v)
        cw.wait()

        zero = jnp.zeros((LANES,), jnp.float32)
        daccs = [zero] * DOT_UNROLL

        def dot_body(j, daccs):
            daccs = list(daccs)
            for u in range(DOT_UNROLL):
                off = j * (DOT_UNROLL * LANES) + u * LANES
                daccs[u] = daccs[u] + row_v[pl.ds(off, LANES)] * w_v[pl.ds(off, LANES)]
            return tuple(daccs)

        daccs = lax.fori_loop(0, N_EMBED // (DOT_UNROLL * LANES), dot_body,
                              tuple(daccs))
        dacc = (daccs[0] + daccs[1]) + (daccs[2] + daccs[3])
        lanes = [dacc[i] for i in range(LANES)]
        while len(lanes) > 1:
            lanes = [lanes[i] + lanes[i + 1] for i in range(0, len(lanes), 2)]
        logit = lanes[0]

        tmp_v[...] = jnp.where(iota == w, logit, 0.0)
        pltpu.sync_copy(tmp_v, stage_hbm.at[w])

    plsc.subcore_barrier()

    @pl.when(w == 0)
    def _():
        pltpu.sync_copy(stage_hbm, big_v)
        g = big_v[0, :] + w_v[pl.ds(N_EMBED, LANES)]
        for i in range(1, NSEQ):
            g = g + big_v[i, :]
        res_v[...] = g
        pltpu.sync_copy(res_v.at[pl.ds(0, NSEQ)], out_hbm)


@jax.jit
def _mc_head(ids, hid, wb):
    mesh = plsc.VectorSubcoreMesh(core_axis_name="c", subcore_axis_name="s",
                                  num_cores=1)
    f = functools.partial(
        pl.kernel,
        mesh=mesh,
        out_type=(jax.ShapeDtypeStruct((NSEQ,), jnp.float32),
                  jax.ShapeDtypeStruct((NSEQ, LANES), jnp.float32)),
        scratch_types=[
            pltpu.VMEM((SEQ,), jnp.int32),
            pltpu.VMEM((N_EMBED,), jnp.float32),
            pltpu.VMEM((N_EMBED + LANES,), jnp.float32),
            pltpu.VMEM((LANES,), jnp.float32),
            pltpu.VMEM((LANES,), jnp.float32),
            pltpu.VMEM((NSEQ, LANES), jnp.float32),
            pltpu.SemaphoreType.DMA,
            pltpu.SemaphoreType.DMA,
            pltpu.SemaphoreType.DMA,
        ],
    )(_mc_head_body)
    return f(ids, hid, wb)


def kernel(hidden, inputs, W, b):
    n_batch, n_choices = inputs.shape[0], inputs.shape[1]
    ids = inputs.reshape(NSEQ, SEQ, 2)[:, :, 0]
    hid = hidden.reshape(NSEQ * SEQ, N_EMBED)
    wb = jnp.concatenate([W[:, 0], jnp.broadcast_to(b, (LANES,))])
    out, _stage = _mc_head(ids, hid, wb)
    return out.reshape(n_batch, n_choices)


# speedup vs baseline: 1.8526x; 1.0313x over previous
import functools

import jax
import jax.numpy as jnp
from jax import lax
from jax.experimental import pallas as pl
from jax.experimental.pallas import tpu as pltpu
from jax.experimental.pallas import tpu_sc as plsc

N_EMBED = 768
CLF_TOKEN = 40480
SEQ = 2048
NSEQ = 8
LANES = 16
SCAN_UNROLL = 4
DOT_UNROLL = 4


def _mc_head_body(ids_hbm, hid_hbm, wb_hbm, out_hbm,
                  ids_v, row_v, w_v, tmp_v,
                  sem0, sem1, sem2):
    w = lax.axis_index("s")
    iota = lax.iota(jnp.int32, LANES)

    HALF = SEQ // 2

    @pl.when(w < NSEQ)
    def _():
        c0 = pltpu.async_copy(ids_hbm.at[w, pl.ds(0, HALF)],
                              ids_v.at[pl.ds(0, HALF)], sem0)
        c1 = pltpu.async_copy(ids_hbm.at[w, pl.ds(HALF, HALF)],
                              ids_v.at[pl.ds(HALF, HALF)], sem1)
        cw = pltpu.async_copy(wb_hbm, w_v, sem2)

        iota_f = iota.astype(jnp.float32)
        neg1 = jnp.full((LANES,), -1.0, jnp.float32)

        def make_scan(base):
            def scan_body(i, accs):
                accs = list(accs)
                for u in range(SCAN_UNROLL):
                    off = base + i * (SCAN_UNROLL * LANES) + u * LANES
                    chunk = ids_v[pl.ds(off, LANES)]
                    idxs = iota_f + jnp.float32(off)
                    accs[u] = jnp.maximum(
                        accs[u], jnp.where(chunk == CLF_TOKEN, idxs, -1.0))
                return tuple(accs)
            return scan_body

        n_it = HALF // (SCAN_UNROLL * LANES)
        c0.wait()
        accs = lax.fori_loop(0, n_it, make_scan(0), (neg1,) * SCAN_UNROLL)
        c1.wait()
        accs = lax.fori_loop(0, n_it, make_scan(HALF), accs)
        acc = jnp.maximum(jnp.maximum(accs[0], accs[1]),
                          jnp.maximum(accs[2], accs[3]))
        pos_f = acc[0]
        for i in range(1, LANES):
            pos_f = jnp.maximum(pos_f, acc[i])
        pos = pos_f.astype(jnp.int32)

        pltpu.sync_copy(hid_hbm.at[w * SEQ + pos], row_v)
        cw.wait()

        zero = jnp.zeros((LANES,), jnp.float32)
        daccs = [zero] * DOT_UNROLL

        def dot_body(j, daccs):
            daccs = list(daccs)
            for u in range(DOT_UNROLL):
                off = j * (DOT_UNROLL * LANES) + u * LANES
                daccs[u] = daccs[u] + row_v[pl.ds(off, LANES)] * w_v[pl.ds(off, LANES)]
            return tuple(daccs)

        daccs = lax.fori_loop(0, N_EMBED // (DOT_UNROLL * LANES), dot_body,
                              tuple(daccs))
        dacc = (daccs[0] + daccs[1]) + (daccs[2] + daccs[3])
        lanes = [dacc[i] for i in range(LANES)]
        while len(lanes) > 1:
            lanes = [lanes[i] + lanes[i + 1] for i in range(0, len(lanes), 2)]
        logit = lanes[0]

        logit = logit + w_v[pl.ds(N_EMBED, LANES)][0]
        tmp_v[...] = jnp.full((LANES,), logit, jnp.float32)
        pltpu.sync_copy(tmp_v, out_hbm.at[w])


@jax.jit
def _mc_head(ids, hid, wb):
    mesh = plsc.VectorSubcoreMesh(core_axis_name="c", subcore_axis_name="s",
                                  num_cores=1)
    f = functools.partial(
        pl.kernel,
        mesh=mesh,
        out_type=jax.ShapeDtypeStruct((NSEQ, LANES), jnp.float32),
        scratch_types=[
            pltpu.VMEM((SEQ,), jnp.int32),
            pltpu.VMEM((N_EMBED,), jnp.float32),
            pltpu.VMEM((N_EMBED + LANES,), jnp.float32),
            pltpu.VMEM((LANES,), jnp.float32),
            pltpu.SemaphoreType.DMA,
            pltpu.SemaphoreType.DMA,
            pltpu.SemaphoreType.DMA,
        ],
    )(_mc_head_body)
    return f(ids, hid, wb)


def kernel(hidden, inputs, W, b):
    n_batch, n_choices = inputs.shape[0], inputs.shape[1]
    ids = inputs.reshape(NSEQ, SEQ, 2)[:, :, 0]
    hid = hidden.reshape(NSEQ * SEQ, N_EMBED)
    wb = jnp.concatenate([W[:, 0], jnp.broadcast_to(b, (LANES,))])
    out = _mc_head(ids, hid, wb)
    return out[:, 0].reshape(n_batch, n_choices)
